# baseline (speedup 1.0000x reference)
import functools

import jax
import jax.numpy as jnp
from jax import lax
from jax.experimental import pallas as pl
from jax.experimental.pallas import tpu as pltpu
from jax.experimental.pallas import tpu_sc as plsc

F32 = jnp.float32

N = 10000
E = 320000
B = 16
NODE_IN = 128
EDGE_IN = 16
NODE_OUT = 128
EDGE_OUT = 128
GLOBAL_IN = 64

NB = 2048
EB = 2000
N_EBLK = E // EB
NC = 2
NS = 16
NW = NC * NS
EW = E // NW
C = 80
NCH = EW // C
NP = 10240
ROWS_T = NP // NS
AGG_W = EDGE_OUT + 16


def _ka_body(x_ref, w1_ref, be_ref, wk_ref, bk_ref, wc_ref, wn_ref, bn_ref,
             xe_ref, k2_ref, ft_ref):
    xb = x_ref[...]
    xe_ref[...] = jnp.dot(xb, w1_ref[...], preferred_element_type=F32) + be_ref[...]
    key = jnp.dot(xb, wk_ref[...], preferred_element_type=F32) + bk_ref[...]
    k2_ref[...] = lax.dot_general(key, wc_ref[...], (((1,), (1,)), ((), ())),
                                  preferred_element_type=F32)
    ft_ref[...] = jnp.dot(xb, wn_ref[...], preferred_element_type=F32) + bn_ref[...]


def _node_precompute(x, W1, be, Wk, bk, Wc, Wn, bn):
    full = lambda s: pl.BlockSpec(s, lambda i: (0, 0))
    return pl.pallas_call(
        _ka_body,
        grid=(NP // NB,),
        in_specs=[
            pl.BlockSpec((NB, NODE_IN), lambda i: (i, 0)),
            full((NODE_IN, EDGE_OUT)), full((1, EDGE_OUT)),
            full((NODE_IN, NODE_OUT)), full((1, NODE_OUT)),
            full((EDGE_OUT, NODE_OUT)),
            full((NODE_IN, NODE_OUT)), full((1, NODE_OUT)),
        ],
        out_specs=[
            pl.BlockSpec((NB, EDGE_OUT), lambda i: (i, 0)),
            pl.BlockSpec((NB, NODE_OUT), lambda i: (i, 0)),
            pl.BlockSpec((NB, NODE_OUT), lambda i: (i, 0)),
        ],
        out_shape=[
            jax.ShapeDtypeStruct((NP, EDGE_OUT), F32),
            jax.ShapeDtypeStruct((NP, NODE_OUT), F32),
            jax.ShapeDtypeStruct((NP, NODE_OUT), F32),
        ],
    )(x, W1, be, Wk, bk, Wc, Wn, bn)


NBUF = 5
GC = NBUF * C
NG = EW // GC


def _make_s1():
    mesh = plsc.VectorSubcoreMesh(core_axis_name="c", subcore_axis_name="s")

    @functools.partial(
        pl.kernel,
        out_type=(jax.ShapeDtypeStruct((E, EDGE_OUT), F32),
                  jax.ShapeDtypeStruct((E, NODE_OUT), F32)),
        mesh=mesh,
        scratch_types=(
            [pltpu.VMEM((GC,), jnp.int32),
             pltpu.VMEM((GC,), jnp.int32),
             pltpu.VMEM((NBUF, C, EDGE_OUT), F32),
             pltpu.VMEM((NBUF, C, NODE_OUT), F32)]
            + [pltpu.SemaphoreType.DMA] * (2 * NBUF + 1)
        ),
    )
    def s1(s_hbm, r_hbm, xe_hbm, k2_hbm, xs_out, k2r_out,
           sidx_v, ridx_v, xs_v, k2_v, *sems):
        gsem = sems[:NBUF]
        wsem = sems[NBUF:2 * NBUF]
        isem = sems[2 * NBUF]
        wid = lax.axis_index("s") * NC + lax.axis_index("c")
        base0 = wid * EW

        @pl.loop(0, NG)
        def _grp(g):
            i0 = base0 + g * GC

            pltpu.async_copy(s_hbm.at[pl.ds(i0, GC)], sidx_v, isem)
            pltpu.async_copy(r_hbm.at[pl.ds(i0, GC)], ridx_v, isem)

            @pl.when(g > 0)
            def _drain():
                p0 = i0 - GC
                for b in range(NBUF):
                    pb = p0 + b * C
                    pltpu.make_async_copy(
                        xs_v.at[b], xs_out.at[pl.ds(pb, C)], wsem[b]).wait()
                    pltpu.make_async_copy(
                        k2_v.at[b], k2r_out.at[pl.ds(pb, C)], wsem[b]).wait()

            pltpu.make_async_copy(s_hbm.at[pl.ds(i0, GC)], sidx_v, isem).wait()
            pltpu.make_async_copy(r_hbm.at[pl.ds(i0, GC)], ridx_v, isem).wait()
            for b in range(NBUF):
                pltpu.async_copy(xe_hbm.at[sidx_v.at[pl.ds(b * C, C)]],
                                 xs_v.at[b], gsem[b])
                pltpu.async_copy(k2_hbm.at[ridx_v.at[pl.ds(b * C, C)]],
                                 k2_v.at[b], gsem[b])
            for b in range(NBUF):
                base = i0 + b * C
                pltpu.make_async_copy(xe_hbm.at[sidx_v.at[pl.ds(b * C, C)]],
                                      xs_v.at[b], gsem[b]).wait()
                pltpu.make_async_copy(k2_hbm.at[ridx_v.at[pl.ds(b * C, C)]],
                                      k2_v.at[b], gsem[b]).wait()
                pltpu.async_copy(xs_v.at[b], xs_out.at[pl.ds(base, C)], wsem[b])
                pltpu.async_copy(k2_v.at[b], k2r_out.at[pl.ds(base, C)], wsem[b])

        p0 = base0 + (NG - 1) * GC
        for b in range(NBUF):
            pb = p0 + b * C
            pltpu.make_async_copy(
                xs_v.at[b], xs_out.at[pl.ds(pb, C)], wsem[b]).wait()
            pltpu.make_async_copy(
                k2_v.at[b], k2r_out.at[pl.ds(pb, C)], wsem[b]).wait()

    return s1


def _kc_body(xs_ref, ea_ref, k2r_ref, w2_ref, e_ref, lg_ref, bm_ref):
    ev = jnp.maximum(
        xs_ref[...] + jnp.dot(ea_ref[...], w2_ref[...], preferred_element_type=F32),
        0.0)
    e_ref[...] = ev
    lg = jnp.sum(ev * k2r_ref[...], axis=1)
    lg_ref[...] = lg[None, None, :]

    @pl.when(pl.program_id(0) == 0)
    def _init():
        bm_ref[...] = jnp.full((8, 128), -jnp.inf, dtype=F32)

    bm_ref[...] = jnp.maximum(bm_ref[...], jnp.max(lg))


def _edge_compute(xs, ea, k2r, W2):
    return pl.pallas_call(
        _kc_body,
        grid=(N_EBLK,),
        in_specs=[
            pl.BlockSpec((EB, EDGE_OUT), lambda i: (i, 0)),
            pl.BlockSpec((EB, EDGE_IN), lambda i: (i, 0)),
            pl.BlockSpec((EB, NODE_OUT), lambda i: (i, 0)),
            pl.BlockSpec((EDGE_IN, EDGE_OUT), lambda i: (0, 0)),
        ],
        out_specs=[
            pl.BlockSpec((EB, EDGE_OUT), lambda i: (i, 0)),
            pl.BlockSpec((1, 1, EB), lambda i: (i, 0, 0)),
            pl.BlockSpec((8, 128), lambda i: (0, 0)),
        ],
        out_shape=[
            jax.ShapeDtypeStruct((E, EDGE_OUT), F32),
            jax.ShapeDtypeStruct((N_EBLK, 1, EB), F32),
            jax.ShapeDtypeStruct((8, 128), F32),
        ],
    )(xs, ea, k2r, W2)


def _make_s2():
    mesh = plsc.VectorSubcoreMesh(core_axis_name="c", subcore_axis_name="s")

    @functools.partial(
        pl.kernel,
        out_type=(jax.ShapeDtypeStruct((NC * NP, EDGE_OUT), F32),
                  jax.ShapeDtypeStruct((NC * NP,), F32)),
        mesh=mesh,
        scratch_types=(
            [pltpu.VMEM((NBUF, C), jnp.int32),
             pltpu.VMEM((2, C, EDGE_OUT), F32),
             pltpu.VMEM((NBUF, C), F32),
             pltpu.VMEM((NBUF, C), F32),
             pltpu.VMEM((128,), F32),
             pltpu.VMEM_SHARED((NP, EDGE_OUT), F32),
             pltpu.VMEM_SHARED((NP,), F32)]
            + [pltpu.SemaphoreType.DMA] * (NBUF + 2)
        ),
    )
    def s2(r_hbm, e_hbm, lg_hbm, gm_hbm, z128, zflat, aggp, denp,
           ridx_v, ebuf, lgv, exv, gv, agg_sh, den_sh, *sems):
        asem = sems[:NBUF]
        lsem = sems[NBUF:]
        cid = lax.axis_index("c")
        sid = lax.axis_index("s")
        wid = sid * NC + cid
        rows0 = sid * ROWS_T
        pltpu.sync_copy(z128, agg_sh.at[pl.ds(rows0, ROWS_T)])
        pltpu.sync_copy(zflat, den_sh.at[pl.ds(rows0, ROWS_T)])
        pltpu.sync_copy(gm_hbm.at[0], gv)
        gmax = gv[pl.ds(0, 16)][0]
        plsc.subcore_barrier()

        base0 = wid * EW

        @pl.loop(0, NG)
        def _grp(g):
            i0 = base0 + g * GC

            for b in range(NBUF):
                base = i0 + b * C
                pltpu.async_copy(r_hbm.at[pl.ds(base, C)], ridx_v.at[b],
                                 asem[b])
                pltpu.async_copy(lg_hbm.at[pl.ds(base, C)], lgv.at[b],
                                 asem[b])
            for b in range(2):
                base = i0 + b * C
                pltpu.async_copy(e_hbm.at[pl.ds(base, C)], ebuf.at[b],
                                 lsem[b])
            for b in range(NBUF):
                base = i0 + b * C
                pltpu.make_async_copy(r_hbm.at[pl.ds(base, C)],
                                      ridx_v.at[b], asem[b]).wait()
                pltpu.make_async_copy(lg_hbm.at[pl.ds(base, C)],
                                      lgv.at[b], asem[b]).wait()
                pltpu.make_async_copy(e_hbm.at[pl.ds(base, C)],
                                      ebuf.at[b % 2], lsem[b % 2]).wait()
                eb = ebuf.at[b % 2]

                @pl.loop(0, C // 16)
                def _w(j, _b=b, _eb=eb):
                    lv = lgv[_b, pl.ds(j * 16, 16)]
                    ev = jnp.exp(lv - gmax)
                    exv[_b, pl.ds(j * 16, 16)] = ev
                    for l in range(16):
                        s = ev[l]
                        row = j * 16 + l
                        for k in range(EDGE_OUT // 16):
                            sl = pl.ds(k * 16, 16)
                            _eb[row, sl] = _eb[row, sl] * s

                pltpu.sync_copy(eb, agg_sh.at[ridx_v.at[b]], add=True)
                pltpu.sync_copy(exv.at[b], den_sh.at[ridx_v.at[b]], add=True)
                if b + 2 < NBUF:
                    nbase = i0 + (b + 2) * C
                    pltpu.async_copy(e_hbm.at[pl.ds(nbase, C)],
                                     ebuf.at[b % 2], lsem[b % 2])

        plsc.subcore_barrier()
        pltpu.sync_copy(agg_sh.at[pl.ds(rows0, ROWS_T)],
                        aggp.at[pl.ds(cid * NP + rows0, ROWS_T)])
        pltpu.sync_copy(den_sh.at[pl.ds(rows0, ROWS_T)],
                        denp.at[pl.ds(cid * NP + rows0, ROWS_T)])

    return s2


def _ke_body(ft_ref, ag_ref, dn_ref, bt_ref, gl_ref, w2f_ref, w2a_ref, w2g_ref,
             b2_ref, wg1_ref, wg2_ref, bg_ref, nodes_ref, ps_ref, ct_ref,
             u_ref):
    dn = dn_ref[0] + dn_ref[1]
    agg = (ag_ref[0] + ag_ref[1]) / (dn + 1e-16)
    b = bt_ref[0, 0, :]
    oh = (b[:, None] == lax.broadcasted_iota(jnp.int32, (NB, B), 1)).astype(F32)
    gb = jnp.dot(oh, gl_ref[...], preferred_element_type=F32)
    nodes = jnp.maximum(
        jnp.dot(ft_ref[...], w2f_ref[...], preferred_element_type=F32)
        + jnp.dot(agg, w2a_ref[...], preferred_element_type=F32)
        + jnp.dot(gb, w2g_ref[...], preferred_element_type=F32)
        + b2_ref[...], 0.0)
    nodes_ref[...] = nodes

    @pl.when(pl.program_id(0) == 0)
    def _init():
        ps_ref[...] = jnp.zeros((B, NODE_OUT), F32)
        ct_ref[...] = jnp.zeros((B, 128), F32)

    ps_ref[...] += lax.dot_general(oh, nodes, (((0,), (0,)), ((), ())),
                                   preferred_element_type=F32)
    cnt = jnp.sum(oh, axis=0)
    ct_ref[...] += jnp.broadcast_to(cnt[:, None], (B, 128))

    @pl.when(pl.program_id(0) == NP // NB - 1)
    def _global_model():
        mean = ps_ref[...] / jnp.maximum(ct_ref[...], 1.0)
        u_ref[...] = jnp.maximum(
            jnp.dot(gl_ref[...], wg1_ref[...], preferred_element_type=F32)
            + jnp.dot(mean, wg2_ref[...], preferred_element_type=F32)
            + bg_ref[...], 0.0)


def _node_update(feat, aggp, denp, batch3, glob, W2f, W2a, W2g, b2,
                 Wg1, Wg2, bg):
    return pl.pallas_call(
        _ke_body,
        grid=(NP // NB,),
        in_specs=[
            pl.BlockSpec((NB, NODE_OUT), lambda i: (i, 0)),
            pl.BlockSpec((NC, NB, EDGE_OUT), lambda i: (0, i, 0)),
            pl.BlockSpec((NC, NB, 1), lambda i: (0, i, 0)),
            pl.BlockSpec((1, 1, NB), lambda i: (i, 0, 0)),
            pl.BlockSpec((B, GLOBAL_IN), lambda i: (0, 0)),
            pl.BlockSpec((NODE_OUT, NODE_OUT), lambda i: (0, 0)),
            pl.BlockSpec((EDGE_OUT, NODE_OUT), lambda i: (0, 0)),
            pl.BlockSpec((GLOBAL_IN, NODE_OUT), lambda i: (0, 0)),
            pl.BlockSpec((1, NODE_OUT), lambda i: (0, 0)),
            pl.BlockSpec((GLOBAL_IN, GLOBAL_IN), lambda i: (0, 0)),
            pl.BlockSpec((NODE_OUT, GLOBAL_IN), lambda i: (0, 0)),
            pl.BlockSpec((1, GLOBAL_IN), lambda i: (0, 0)),
        ],
        out_specs=[
            pl.BlockSpec((NB, NODE_OUT), lambda i: (i, 0)),
            pl.BlockSpec((B, NODE_OUT), lambda i: (0, 0)),
            pl.BlockSpec((B, 128), lambda i: (0, 0)),
            pl.BlockSpec((B, GLOBAL_IN), lambda i: (0, 0)),
        ],
        out_shape=[
            jax.ShapeDtypeStruct((NP, NODE_OUT), F32),
            jax.ShapeDtypeStruct((B, NODE_OUT), F32),
            jax.ShapeDtypeStruct((B, 128), F32),
            jax.ShapeDtypeStruct((B, GLOBAL_IN), F32),
        ],
    )(feat, aggp, denp, batch3, glob, W2f, W2a, W2g, b2, Wg1, Wg2, bg)


def kernel(x, edge_index, edge_attr, glob, batch,
           W_edge, b_edge, W_code, b_code, W_key, b_key,
           W_node, b_node, W_node2, b_node2, W_glob, b_glob):
    S = edge_index[0].astype(jnp.int32)
    R = edge_index[1].astype(jnp.int32)
    xp = jnp.pad(x, ((0, NP - N), (0, 0)))

    W1 = W_edge[:NODE_IN]
    W2 = W_edge[NODE_IN:]
    xe, key2, feat = _node_precompute(
        xp, W1, b_edge[None, :], W_key, b_key[None, :], W_code,
        W_node, b_node[None, :])

    xs, k2r = _make_s1()(S, R, xe, key2)

    e, lg3, gmax = _edge_compute(xs, edge_attr, k2r, W2)
    lg1 = lg3.reshape(E)

    z128 = jnp.zeros((ROWS_T, EDGE_OUT), F32)
    zflat = jnp.zeros((ROWS_T,), F32)
    aggp, denp = _make_s2()(R, e, lg1, gmax, z128, zflat)
    aggp = aggp.reshape(NC, NP, EDGE_OUT)
    denp = denp.reshape(NC, NP, 1)

    batchp = jnp.pad(batch.astype(jnp.int32), (0, NP - N),
                     constant_values=B)
    batch3 = batchp.reshape(NP // NB, 1, NB)
    W2f = W_node2[:NODE_OUT]
    W2a = W_node2[NODE_OUT:2 * NODE_OUT]
    W2g = W_node2[2 * NODE_OUT:]
    Wg1 = W_glob[:GLOBAL_IN]
    Wg2 = W_glob[GLOBAL_IN:]
    nodes, _, _, u = _node_update(feat, aggp, denp, batch3, glob,
                                  W2f, W2a, W2g, b_node2[None, :],
                                  Wg1, Wg2, b_glob[None, :])

    return nodes[:N], e, u


# speedup vs baseline: 1.0229x; 1.0229x over previous
import functools

import jax
import jax.numpy as jnp
from jax import lax
from jax.experimental import pallas as pl
from jax.experimental.pallas import tpu as pltpu
from jax.experimental.pallas import tpu_sc as plsc

F32 = jnp.float32

N = 10000
E = 320000
B = 16
NODE_IN = 128
EDGE_IN = 16
NODE_OUT = 128
EDGE_OUT = 128
GLOBAL_IN = 64

NB = 2048
EB = 2000
N_EBLK = E // EB
NC = 2
NS = 16
NW = NC * NS
EW = E // NW
C = 80
NCH = EW // C
NP = 10240
ROWS_T = NP // NS
AGG_W = EDGE_OUT + 16


def _ka_body(x_ref, w1_ref, be_ref, wk_ref, bk_ref, wc_ref, wn_ref, bn_ref,
             xe_ref, k2_ref, ft_ref):
    xb = x_ref[...]
    xe_ref[...] = jnp.dot(xb, w1_ref[...], preferred_element_type=F32) + be_ref[...]
    key = jnp.dot(xb, wk_ref[...], preferred_element_type=F32) + bk_ref[...]
    k2_ref[...] = lax.dot_general(key, wc_ref[...], (((1,), (1,)), ((), ())),
                                  preferred_element_type=F32)
    ft_ref[...] = jnp.dot(xb, wn_ref[...], preferred_element_type=F32) + bn_ref[...]


def _node_precompute(x, W1, be, Wk, bk, Wc, Wn, bn):
    full = lambda s: pl.BlockSpec(s, lambda i: (0, 0))
    return pl.pallas_call(
        _ka_body,
        grid=(NP // NB,),
        in_specs=[
            pl.BlockSpec((NB, NODE_IN), lambda i: (i, 0)),
            full((NODE_IN, EDGE_OUT)), full((1, EDGE_OUT)),
            full((NODE_IN, NODE_OUT)), full((1, NODE_OUT)),
            full((EDGE_OUT, NODE_OUT)),
            full((NODE_IN, NODE_OUT)), full((1, NODE_OUT)),
        ],
        out_specs=[
            pl.BlockSpec((NB, EDGE_OUT), lambda i: (i, 0)),
            pl.BlockSpec((NB, NODE_OUT), lambda i: (i, 0)),
            pl.BlockSpec((NB, NODE_OUT), lambda i: (i, 0)),
        ],
        out_shape=[
            jax.ShapeDtypeStruct((NP, EDGE_OUT), F32),
            jax.ShapeDtypeStruct((NP, NODE_OUT), F32),
            jax.ShapeDtypeStruct((NP, NODE_OUT), F32),
        ],
    )(x, W1, be, Wk, bk, Wc, Wn, bn)


NBUF = 5
GC = NBUF * C
NG = EW // GC


def _make_s1():
    mesh = plsc.VectorSubcoreMesh(core_axis_name="c", subcore_axis_name="s")

    @functools.partial(
        pl.kernel,
        out_type=(jax.ShapeDtypeStruct((E, EDGE_OUT), F32),
                  jax.ShapeDtypeStruct((E, NODE_OUT), F32)),
        mesh=mesh,
        scratch_types=(
            [pltpu.VMEM((GC,), jnp.int32),
             pltpu.VMEM((GC,), jnp.int32),
             pltpu.VMEM((NBUF, C, EDGE_OUT), F32),
             pltpu.VMEM((NBUF, C, NODE_OUT), F32)]
            + [pltpu.SemaphoreType.DMA] * (2 * NBUF + 1)
        ),
    )
    def s1(s_hbm, r_hbm, xe_hbm, k2_hbm, xs_out, k2r_out,
           sidx_v, ridx_v, xs_v, k2_v, *sems):
        gsem = sems[:NBUF]
        wsem = sems[NBUF:2 * NBUF]
        isem = sems[2 * NBUF]
        wid = lax.axis_index("s") * NC + lax.axis_index("c")
        base0 = wid * EW

        @pl.loop(0, NG)
        def _grp(g):
            i0 = base0 + g * GC

            pltpu.async_copy(s_hbm.at[pl.ds(i0, GC)], sidx_v, isem)
            pltpu.async_copy(r_hbm.at[pl.ds(i0, GC)], ridx_v, isem)

            @pl.when(g > 0)
            def _drain():
                p0 = i0 - GC
                for b in range(NBUF):
                    pb = p0 + b * C
                    pltpu.make_async_copy(
                        xs_v.at[b], xs_out.at[pl.ds(pb, C)], wsem[b]).wait()
                    pltpu.make_async_copy(
                        k2_v.at[b], k2r_out.at[pl.ds(pb, C)], wsem[b]).wait()

            pltpu.make_async_copy(s_hbm.at[pl.ds(i0, GC)], sidx_v, isem).wait()
            pltpu.make_async_copy(r_hbm.at[pl.ds(i0, GC)], ridx_v, isem).wait()
            for b in range(NBUF):
                pltpu.async_copy(xe_hbm.at[sidx_v.at[pl.ds(b * C, C)]],
                                 xs_v.at[b], gsem[b])
                pltpu.async_copy(k2_hbm.at[ridx_v.at[pl.ds(b * C, C)]],
                                 k2_v.at[b], gsem[b])
            for b in range(NBUF):
                base = i0 + b * C
                pltpu.make_async_copy(xe_hbm.at[sidx_v.at[pl.ds(b * C, C)]],
                                      xs_v.at[b], gsem[b]).wait()
                pltpu.make_async_copy(k2_hbm.at[ridx_v.at[pl.ds(b * C, C)]],
                                      k2_v.at[b], gsem[b]).wait()
                pltpu.async_copy(xs_v.at[b], xs_out.at[pl.ds(base, C)], wsem[b])
                pltpu.async_copy(k2_v.at[b], k2r_out.at[pl.ds(base, C)], wsem[b])

        p0 = base0 + (NG - 1) * GC
        for b in range(NBUF):
            pb = p0 + b * C
            pltpu.make_async_copy(
                xs_v.at[b], xs_out.at[pl.ds(pb, C)], wsem[b]).wait()
            pltpu.make_async_copy(
                k2_v.at[b], k2r_out.at[pl.ds(pb, C)], wsem[b]).wait()

    return s1


def _kc_body(xs_ref, ea_ref, k2r_ref, w2_ref, e_ref, lg_ref):
    ev = jnp.maximum(
        xs_ref[...] + jnp.dot(ea_ref[...], w2_ref[...], preferred_element_type=F32),
        0.0)
    e_ref[...] = ev
    lg_ref[...] = jnp.sum(ev * k2r_ref[...], axis=1)[None, None, :]


def _edge_compute(xs, ea, k2r, W2):
    return pl.pallas_call(
        _kc_body,
        grid=(N_EBLK,),
        in_specs=[
            pl.BlockSpec((EB, EDGE_OUT), lambda i: (i, 0)),
            pl.BlockSpec((EB, EDGE_IN), lambda i: (i, 0)),
            pl.BlockSpec((EB, NODE_OUT), lambda i: (i, 0)),
            pl.BlockSpec((EDGE_IN, EDGE_OUT), lambda i: (0, 0)),
        ],
        out_specs=[
            pl.BlockSpec((EB, EDGE_OUT), lambda i: (i, 0)),
            pl.BlockSpec((1, 1, EB), lambda i: (i, 0, 0)),
        ],
        out_shape=[
            jax.ShapeDtypeStruct((E, EDGE_OUT), F32),
            jax.ShapeDtypeStruct((N_EBLK, 1, EB), F32),
        ],
    )(xs, ea, k2r, W2)


def _kd1_body(lg_ref, out_ref):
    out_ref[...] = jnp.full((8, 128), jnp.max(lg_ref[...]), dtype=F32)


def _global_max(lg3):
    return pl.pallas_call(
        _kd1_body,
        grid=(1,),
        in_specs=[pl.BlockSpec((N_EBLK, 1, EB), lambda i: (0, 0, 0))],
        out_specs=pl.BlockSpec((8, 128), lambda i: (0, 0)),
        out_shape=jax.ShapeDtypeStruct((8, 128), F32),
    )(lg3)


def _make_s2():
    mesh = plsc.VectorSubcoreMesh(core_axis_name="c", subcore_axis_name="s")

    @functools.partial(
        pl.kernel,
        out_type=(jax.ShapeDtypeStruct((NC * NP, EDGE_OUT), F32),
                  jax.ShapeDtypeStruct((NC * NP,), F32)),
        mesh=mesh,
        scratch_types=(
            [pltpu.VMEM((NBUF, C), jnp.int32),
             pltpu.VMEM((2, C, EDGE_OUT), F32),
             pltpu.VMEM((NBUF, C), F32),
             pltpu.VMEM((NBUF, C), F32),
             pltpu.VMEM((128,), F32),
             pltpu.VMEM_SHARED((NP, EDGE_OUT), F32),
             pltpu.VMEM_SHARED((NP,), F32)]
            + [pltpu.SemaphoreType.DMA] * (NBUF + 2)
        ),
    )
    def s2(r_hbm, e_hbm, lg_hbm, gm_hbm, z128, zflat, aggp, denp,
           ridx_v, ebuf, lgv, exv, gv, agg_sh, den_sh, *sems):
        asem = sems[:NBUF]
        lsem = sems[NBUF:]
        cid = lax.axis_index("c")
        sid = lax.axis_index("s")
        wid = sid * NC + cid
        rows0 = sid * ROWS_T
        pltpu.sync_copy(z128, agg_sh.at[pl.ds(rows0, ROWS_T)])
        pltpu.sync_copy(zflat, den_sh.at[pl.ds(rows0, ROWS_T)])
        pltpu.sync_copy(gm_hbm.at[0], gv)
        gmax = gv[pl.ds(0, 16)][0]
        plsc.subcore_barrier()

        base0 = wid * EW

        @pl.loop(0, NG)
        def _grp(g):
            i0 = base0 + g * GC

            for b in range(NBUF):
                base = i0 + b * C
                pltpu.async_copy(r_hbm.at[pl.ds(base, C)], ridx_v.at[b],
                                 asem[b])
                pltpu.async_copy(lg_hbm.at[pl.ds(base, C)], lgv.at[b],
                                 asem[b])
            for b in range(2):
                base = i0 + b * C
                pltpu.async_copy(e_hbm.at[pl.ds(base, C)], ebuf.at[b],
                                 lsem[b])
            for b in range(NBUF):
                base = i0 + b * C
                pltpu.make_async_copy(r_hbm.at[pl.ds(base, C)],
                                      ridx_v.at[b], asem[b]).wait()
                pltpu.make_async_copy(lg_hbm.at[pl.ds(base, C)],
                                      lgv.at[b], asem[b]).wait()
                pltpu.make_async_copy(e_hbm.at[pl.ds(base, C)],
                                      ebuf.at[b % 2], lsem[b % 2]).wait()
                eb = ebuf.at[b % 2]

                @pl.loop(0, C // 16)
                def _w(j, _b=b, _eb=eb):
                    lv = lgv[_b, pl.ds(j * 16, 16)]
                    ev = jnp.exp(lv - gmax)
                    exv[_b, pl.ds(j * 16, 16)] = ev
                    for l in range(16):
                        s = ev[l]
                        row = j * 16 + l
                        for k in range(EDGE_OUT // 16):
                            sl = pl.ds(k * 16, 16)
                            _eb[row, sl] = _eb[row, sl] * s

                pltpu.sync_copy(eb, agg_sh.at[ridx_v.at[b]], add=True)
                pltpu.sync_copy(exv.at[b], den_sh.at[ridx_v.at[b]], add=True)
                if b + 2 < NBUF:
                    nbase = i0 + (b + 2) * C
                    pltpu.async_copy(e_hbm.at[pl.ds(nbase, C)],
                                     ebuf.at[b % 2], lsem[b % 2])

        plsc.subcore_barrier()
        pltpu.sync_copy(agg_sh.at[pl.ds(rows0, ROWS_T)],
                        aggp.at[pl.ds(cid * NP + rows0, ROWS_T)])
        pltpu.sync_copy(den_sh.at[pl.ds(rows0, ROWS_T)],
                        denp.at[pl.ds(cid * NP + rows0, ROWS_T)])

    return s2


def _ke_body(ft_ref, ag_ref, dn_ref, bt_ref, gl_ref, w2f_ref, w2a_ref, w2g_ref,
             b2_ref, wg1_ref, wg2_ref, bg_ref, nodes_ref, ps_ref, ct_ref,
             u_ref):
    dn = dn_ref[0] + dn_ref[1]
    agg = (ag_ref[0] + ag_ref[1]) / (dn + 1e-16)
    b = bt_ref[0, 0, :]
    oh = (b[:, None] == lax.broadcasted_iota(jnp.int32, (NB, B), 1)).astype(F32)
    gb = jnp.dot(oh, gl_ref[...], preferred_element_type=F32)
    nodes = jnp.maximum(
        jnp.dot(ft_ref[...], w2f_ref[...], preferred_element_type=F32)
        + jnp.dot(agg, w2a_ref[...], preferred_element_type=F32)
        + jnp.dot(gb, w2g_ref[...], preferred_element_type=F32)
        + b2_ref[...], 0.0)
    nodes_ref[...] = nodes

    @pl.when(pl.program_id(0) == 0)
    def _init():
        ps_ref[...] = jnp.zeros((B, NODE_OUT), F32)
        ct_ref[...] = jnp.zeros((B, 128), F32)

    ps_ref[...] += lax.dot_general(oh, nodes, (((0,), (0,)), ((), ())),
                                   preferred_element_type=F32)
    cnt = jnp.sum(oh, axis=0)
    ct_ref[...] += jnp.broadcast_to(cnt[:, None], (B, 128))

    @pl.when(pl.program_id(0) == NP // NB - 1)
    def _global_model():
        mean = ps_ref[...] / jnp.maximum(ct_ref[...], 1.0)
        u_ref[...] = jnp.maximum(
            jnp.dot(gl_ref[...], wg1_ref[...], preferred_element_type=F32)
            + jnp.dot(mean, wg2_ref[...], preferred_element_type=F32)
            + bg_ref[...], 0.0)


def _node_update(feat, aggp, denp, batch3, glob, W2f, W2a, W2g, b2,
                 Wg1, Wg2, bg):
    return pl.pallas_call(
        _ke_body,
        grid=(NP // NB,),
        in_specs=[
            pl.BlockSpec((NB, NODE_OUT), lambda i: (i, 0)),
            pl.BlockSpec((NC, NB, EDGE_OUT), lambda i: (0, i, 0)),
            pl.BlockSpec((NC, NB, 1), lambda i: (0, i, 0)),
            pl.BlockSpec((1, 1, NB), lambda i: (i, 0, 0)),
            pl.BlockSpec((B, GLOBAL_IN), lambda i: (0, 0)),
            pl.BlockSpec((NODE_OUT, NODE_OUT), lambda i: (0, 0)),
            pl.BlockSpec((EDGE_OUT, NODE_OUT), lambda i: (0, 0)),
            pl.BlockSpec((GLOBAL_IN, NODE_OUT), lambda i: (0, 0)),
            pl.BlockSpec((1, NODE_OUT), lambda i: (0, 0)),
            pl.BlockSpec((GLOBAL_IN, GLOBAL_IN), lambda i: (0, 0)),
            pl.BlockSpec((NODE_OUT, GLOBAL_IN), lambda i: (0, 0)),
            pl.BlockSpec((1, GLOBAL_IN), lambda i: (0, 0)),
        ],
        out_specs=[
            pl.BlockSpec((NB, NODE_OUT), lambda i: (i, 0)),
            pl.BlockSpec((B, NODE_OUT), lambda i: (0, 0)),
            pl.BlockSpec((B, 128), lambda i: (0, 0)),
            pl.BlockSpec((B, GLOBAL_IN), lambda i: (0, 0)),
        ],
        out_shape=[
            jax.ShapeDtypeStruct((NP, NODE_OUT), F32),
            jax.ShapeDtypeStruct((B, NODE_OUT), F32),
            jax.ShapeDtypeStruct((B, 128), F32),
            jax.ShapeDtypeStruct((B, GLOBAL_IN), F32),
        ],
    )(feat, aggp, denp, batch3, glob, W2f, W2a, W2g, b2, Wg1, Wg2, bg)


def kernel(x, edge_index, edge_attr, glob, batch,
           W_edge, b_edge, W_code, b_code, W_key, b_key,
           W_node, b_node, W_node2, b_node2, W_glob, b_glob):
    S = edge_index[0].astype(jnp.int32)
    R = edge_index[1].astype(jnp.int32)
    xp = jnp.pad(x, ((0, NP - N), (0, 0)))

    W1 = W_edge[:NODE_IN]
    W2 = W_edge[NODE_IN:]
    xe, key2, feat = _node_precompute(
        xp, W1, b_edge[None, :], W_key, b_key[None, :], W_code,
        W_node, b_node[None, :])

    xs, k2r = _make_s1()(S, R, xe, key2)

    e, lg3 = _edge_compute(xs, edge_attr, k2r, W2)
    gmax = _global_max(lg3)
    lg1 = lg3.reshape(E)

    z128 = jnp.zeros((ROWS_T, EDGE_OUT), F32)
    zflat = jnp.zeros((ROWS_T,), F32)
    aggp, denp = _make_s2()(R, e, lg1, gmax, z128, zflat)
    aggp = aggp.reshape(NC, NP, EDGE_OUT)
    denp = denp.reshape(NC, NP, 1)

    batchp = jnp.pad(batch.astype(jnp.int32), (0, NP - N),
                     constant_values=B)
    batch3 = batchp.reshape(NP // NB, 1, NB)
    W2f = W_node2[:NODE_OUT]
    W2a = W_node2[NODE_OUT:2 * NODE_OUT]
    W2g = W_node2[2 * NODE_OUT:]
    Wg1 = W_glob[:GLOBAL_IN]
    Wg2 = W_glob[GLOBAL_IN:]
    nodes, _, _, u = _node_update(feat, aggp, denp, batch3, glob,
                                  W2f, W2a, W2g, b_node2[None, :],
                                  Wg1, Wg2, b_glob[None, :])

    return nodes[:N], e, u
